# baseline (speedup 1.0000x reference)
import functools

import jax
import jax.numpy as jnp
from jax import lax
from jax.experimental import pallas as pl
from jax.experimental.pallas import tpu as pltpu
from jax.experimental.pallas import tpu_sc as plsc

_ROWS = 100000
_EMBED = 64
_NW = 32
_CHUNK = 400
_NCHUNK = _ROWS // _CHUNK
_FULL_ROUNDS = _NCHUNK // _NW
_TAIL = _NCHUNK - _FULL_ROUNDS * _NW


@functools.partial(
    pl.kernel,
    out_type=(
        jax.ShapeDtypeStruct((_ROWS, _EMBED), jnp.float32),
        jax.ShapeDtypeStruct((_ROWS, _EMBED), jnp.float32),
    ),
    mesh=plsc.VectorSubcoreMesh(core_axis_name="c", subcore_axis_name="s"),
    scratch_types=[
        pltpu.VMEM((_CHUNK, _EMBED), jnp.float32),
        pltpu.VMEM((_CHUNK, _EMBED), jnp.float32),
        pltpu.SemaphoreType.DMA,
        pltpu.SemaphoreType.DMA,
        pltpu.SemaphoreType.DMA,
        pltpu.SemaphoreType.DMA,
    ],
)
def _sc_copy(u_in, i_in, u_out, i_out, buf0, buf1, gs0, gs1, ss0, ss1):
    wid = lax.axis_index("s") * 2 + lax.axis_index("c")
    bufs = (buf0, buf1)
    gsem = (gs0, gs1)
    ssem = (ss0, ss1)

    jobs = []
    for t in range(2):
        for r in range(_FULL_ROUNDS):
            jobs.append((t, r))
    n = len(jobs)
    ins = (u_in, i_in)
    outs = (u_out, i_out)

    def chunk_slice(j):
        t, r = jobs[j]
        return ins[t], outs[t], pl.ds((r * _NW + wid) * _CHUNK, _CHUNK)

    def gather(j):
        src, _, sl = chunk_slice(j)
        return pltpu.make_async_copy(src.at[sl], bufs[j % 2], gsem[j % 2])

    def scatter(j):
        _, dst, sl = chunk_slice(j)
        return pltpu.make_async_copy(bufs[j % 2], dst.at[sl], ssem[j % 2])

    gather(0).start()
    gather(1).start()
    for j in range(n):
        gather(j).wait()
        s = scatter(j)
        s.start()
        if j + 2 < n:
            s.wait()
            gather(j + 2).start()
    scatter(n - 2).wait()
    scatter(n - 1).wait()

    def _tail():
        for t in range(2):
            sl = pl.ds((_FULL_ROUNDS * _NW + wid) * _CHUNK, _CHUNK)
            g = pltpu.make_async_copy(ins[t].at[sl], bufs[t], gsem[t])
            g.start()
            g.wait()
            pltpu.make_async_copy(bufs[t], outs[t].at[sl], ssem[t]).start()
        for t in range(2):
            sl = pl.ds((_FULL_ROUNDS * _NW + wid) * _CHUNK, _CHUNK)
            pltpu.make_async_copy(bufs[t], outs[t].at[sl], ssem[t]).wait()

    pl.when(wid < _TAIL)(_tail)


def kernel(user_weight, item_weight):
    return _sc_copy(user_weight, item_weight)


# speedup vs baseline: 1.1029x; 1.1029x over previous
import jax
import jax.numpy as jnp
from jax.experimental import pallas as pl
from jax.experimental.pallas import tpu as pltpu

_ROWS = 100000
_BLK = 10000
_GRID = _ROWS // _BLK


def _copy_kernel(u_in, i_in, u_out, i_out, buf0, buf1, gs0, gs1, ss0, ss1):
    n = pl.program_id(0)
    u_out[...] = u_in[...]

    @pl.when(n == 0)
    def _prologue():
        pltpu.make_async_copy(i_in.at[pl.ds(0, _BLK)], buf0, gs0).start()
        pltpu.make_async_copy(i_in.at[pl.ds(_BLK, _BLK)], buf1, gs1).start()

    def _step(buf, gsem, ssem):
        off = n * _BLK
        sl = pl.ds(off, _BLK)
        pltpu.make_async_copy(i_in.at[sl], buf, gsem).wait()
        out_c = pltpu.make_async_copy(buf, i_out.at[sl], ssem)
        out_c.start()
        out_c.wait()

        @pl.when(n + 2 < _GRID)
        def _next():
            sl2 = pl.ds(off + 2 * _BLK, _BLK)
            pltpu.make_async_copy(i_in.at[sl2], buf, gsem).start()

    pl.when(n % 2 == 0)(lambda: _step(buf0, gs0, ss0))
    pl.when(n % 2 == 1)(lambda: _step(buf1, gs1, ss1))


def kernel(user_weight, item_weight):
    u_spec = pl.BlockSpec((_BLK, 64), lambda n: (n, 0))
    hbm = pl.BlockSpec(memory_space=pltpu.MemorySpace.HBM)
    return pl.pallas_call(
        _copy_kernel,
        grid=(_GRID,),
        out_shape=(
            jax.ShapeDtypeStruct(user_weight.shape, user_weight.dtype),
            jax.ShapeDtypeStruct(item_weight.shape, item_weight.dtype),
        ),
        in_specs=[u_spec, hbm],
        out_specs=(u_spec, hbm),
        scratch_shapes=[
            pltpu.VMEM((_BLK, 64), jnp.float32),
            pltpu.VMEM((_BLK, 64), jnp.float32),
            pltpu.SemaphoreType.DMA,
            pltpu.SemaphoreType.DMA,
            pltpu.SemaphoreType.DMA,
            pltpu.SemaphoreType.DMA,
        ],
        compiler_params=pltpu.CompilerParams(
            vmem_limit_bytes=120_000_000,
        ),
    )(user_weight, item_weight)
